# baseline (speedup 1.0000x reference)
import jax
import jax.numpy as jnp
from jax.experimental import pallas as pl


def _ew_kernel(scores_ref, masks_ref, bias_ref, out_ref):
    out_ref[...] = (scores_ref[...] + bias_ref[0]) * masks_ref[...].astype(jnp.float32)


def kernel(sent_group_scores, sel_sent_emb, sel_sent_masks, group_embs, candi_sent_masks, bias):
    del sel_sent_emb, sel_sent_masks, group_embs
    bias_vec = jnp.reshape(bias, (1,))
    return pl.pallas_call(
        _ew_kernel,
        out_shape=jax.ShapeDtypeStruct(sent_group_scores.shape, jnp.float32),
    )(sent_group_scores, candi_sent_masks, bias_vec)
